# baseline (speedup 1.0000x reference)
import functools

import jax
import jax.numpy as jnp
from jax import lax
from jax.experimental import pallas as pl
from jax.experimental.pallas import tpu as pltpu
from jax.experimental.pallas import tpu_sc as plsc

EPS = 1e-7
D = 32
DPAD = 40
S = 20


def _sc_gather_pack(idx_flat, emb40, cov, n, chunk=320):
    NC, NS = 2, 16
    NW = NC * NS
    per_w = n // NW
    n_chunks = per_w // chunk
    nb = chunk // S
    rows = chunk // 4
    n_out = n // 4
    mesh = plsc.VectorSubcoreMesh(core_axis_name="c", subcore_axis_name="s")

    @functools.partial(
        pl.kernel,
        out_type=(jax.ShapeDtypeStruct((n_out, 128), jnp.float32),
                  jax.ShapeDtypeStruct((n_out, 128), jnp.float32),
                  jax.ShapeDtypeStruct((n_out, 128), jnp.float32),
                  jax.ShapeDtypeStruct((n_out, 128), jnp.float32)),
        mesh=mesh,
        compiler_params=pltpu.CompilerParams(use_tc_tiling_on_sc=False),
        scratch_types=[
            pltpu.VMEM((chunk,), jnp.int32),
            pltpu.VMEM((chunk, DPAD), jnp.float32),
            pltpu.VMEM((chunk, D), jnp.float32),
            pltpu.VMEM((rows, 128), jnp.float32),
            pltpu.VMEM((rows, 128), jnp.float32),
            pltpu.VMEM((rows, 128), jnp.float32),
            pltpu.VMEM((rows, 128), jnp.float32),
            pltpu.SemaphoreType.DMA,
            pltpu.SemaphoreType.DMA,
        ],
    )
    def gather_kernel(idx_hbm, emb_hbm, cov_hbm,
                      h_out, c_out, sf_out, s0_out,
                      idx_v, e_scr, c_scr, h_p, c_p, sf_p, s0_p,
                      sem_e, sem_c):
        wid = lax.axis_index("s") * NC + lax.axis_index("c")
        base = wid * per_w

        @pl.loop(0, n_chunks)
        def _chunk(k):
            start = base + k * chunk
            pltpu.sync_copy(idx_hbm.at[pl.ds(start, chunk)], idx_v)
            ce = pltpu.async_copy(emb_hbm.at[idx_v], e_scr, sem_e)
            cc = pltpu.async_copy(cov_hbm.at[idx_v], c_scr, sem_c)
            ce.wait()
            cc.wait()
            for j in range(chunk):
                r, g = j // 4, 32 * (j % 4)
                h_p[r, pl.ds(g, 16)] = e_scr[j, pl.ds(0, 16)]
                h_p[r, pl.ds(g + 16, 16)] = e_scr[j, pl.ds(16, 16)]
                c_p[r, pl.ds(g, 16)] = c_scr[j, pl.ds(0, 16)]
                c_p[r, pl.ds(g + 16, 16)] = c_scr[j, pl.ds(16, 16)]
            for b in range(nb):
                j0 = b * S
                sa = e_scr[j0, pl.ds(0, 16)]
                sb = e_scr[j0, pl.ds(16, 16)]
                ca = c_scr[j0, pl.ds(0, 16)]
                cb = c_scr[j0, pl.ds(16, 16)]
                for q in range(5):
                    rr = 5 * b + q
                    for g in range(4):
                        sf_p[rr, pl.ds(32 * g, 16)] = sa
                        sf_p[rr, pl.ds(32 * g + 16, 16)] = sb
                        s0_p[rr, pl.ds(32 * g, 16)] = ca
                        s0_p[rr, pl.ds(32 * g + 16, 16)] = cb
            orow = base // 4 + k * rows
            pltpu.sync_copy(h_p, h_out.at[pl.ds(orow, rows)])
            pltpu.sync_copy(c_p, c_out.at[pl.ds(orow, rows)])
            pltpu.sync_copy(sf_p, sf_out.at[pl.ds(orow, rows)])
            pltpu.sync_copy(s0_p, s0_out.at[pl.ds(orow, rows)])

    return gather_kernel(idx_flat, emb40, cov)


def _math_body(h_ref, c_ref, sf_ref, s0_ref, o_ref):
    f32 = jnp.float32
    h = h_ref[...]
    cv = c_ref[...]
    sf = sf_ref[...]
    s0r = s0_ref[...]

    seg = (lax.broadcasted_iota(jnp.int32, (128, 4), 0) // 32
           == lax.broadcasted_iota(jnp.int32, (128, 4), 1)).astype(f32)

    def segsum(x):
        return lax.dot_general(x, seg, (((1,), (0,)), ((), ())),
                               preferred_element_type=f32,
                               precision=lax.Precision.HIGHEST)

    a = 1.0 - EPS
    sig = jnp.where(cv > 0, cv, a * (jnp.exp(cv) - 1.0)) + 1.0
    sig = jnp.maximum(sig, EPS)
    sig0 = jnp.where(s0r > 0, s0r, a * (jnp.exp(s0r) - 1.0)) + 1.0
    sig0 = jnp.maximum(sig0, EPS)
    r0 = 1.0 / sig0

    dots_h = segsum(h * sf)
    hh = segsum(h * h)
    ss = segsum(sf * sf)
    trace = segsum(sig * r0)
    uu_a = segsum(h * h * r0)
    uu_b = segsum(h * sf * r0)
    uu_c = segsum(sf * sf * r0)
    ld_t = segsum(jnp.log(sig))
    ld_s = segsum(jnp.log(sig0))

    t_t = jnp.sqrt(1.0 + hh)
    t_s = jnp.sqrt(1.0 + ss)

    alpha = t_t * t_s - dots_h
    alpha = 1.0 + jnp.maximum(alpha - 1.0, EPS)
    sq = jnp.sqrt(jnp.maximum(alpha * alpha - 1.0, 0.0))
    denom = jnp.maximum(sq, EPS)
    cfac = jnp.log(alpha + sq) / denom
    beta = t_s
    mdot = (-beta * cfac * (dots_h - alpha * ss)
            - (1.0 - beta * t_s) * cfac * (t_t - alpha * t_s))
    scale = mdot / jnp.maximum(beta + 1.0, EPS)
    mu = scale - cfac * alpha
    uu = cfac * cfac * uu_a + 2.0 * cfac * mu * uu_b + mu * mu * uu_c
    kds = 0.5 * (trace + uu - D - (ld_t - ld_s))
    o_ref[...] = kds


def _tc_math(h, c, sf, s0, bB=256, interpret=False):
    R = h.shape[0]
    rB = bB * S // 4
    return pl.pallas_call(
        _math_body,
        grid=(R // rB,),
        in_specs=[pl.BlockSpec((rB, 128), lambda i: (i, 0)),
                  pl.BlockSpec((rB, 128), lambda i: (i, 0)),
                  pl.BlockSpec((rB, 128), lambda i: (i, 0)),
                  pl.BlockSpec((rB, 128), lambda i: (i, 0))],
        out_specs=pl.BlockSpec((rB, 4), lambda i: (i, 0)),
        out_shape=jax.ShapeDtypeStruct((R, 4), jnp.float32),
        interpret=interpret,
    )(h, c, sf, s0)


def kernel(idx, embedding, covariance):
    B, S_ = idx.shape
    n = B * S_
    idx_flat = idx.reshape(-1)
    emb40 = jnp.pad(embedding, ((0, 0), (0, DPAD - (D + 1))))
    h, c, sf, s0 = _sc_gather_pack(idx_flat, emb40, covariance, n)
    kds4 = _tc_math(h, c, sf, s0)
    return kds4.reshape(B, S_)[:, 1:]


# speedup vs baseline: 1.3182x; 1.3182x over previous
import functools

import jax
import jax.numpy as jnp
from jax import lax
from jax.experimental import pallas as pl
from jax.experimental.pallas import tpu as pltpu
from jax.experimental.pallas import tpu_sc as plsc

EPS = 1e-7
D = 32
DPAD = 40
S = 20


def _sc_gather_pack(idx_flat, emb40, cov, n, chunk=320):
    NC, NS = 2, 16
    NW = NC * NS
    per_w = n // NW
    n_chunks = per_w // chunk
    nb = chunk // S
    rows = chunk // 4
    n_out = n // 4
    mesh = plsc.VectorSubcoreMesh(core_axis_name="c", subcore_axis_name="s")

    @functools.partial(
        pl.kernel,
        out_type=(jax.ShapeDtypeStruct((n_out, 128), jnp.float32),
                  jax.ShapeDtypeStruct((n_out, 128), jnp.float32),
                  jax.ShapeDtypeStruct((n_out, 128), jnp.float32),
                  jax.ShapeDtypeStruct((n_out, 128), jnp.float32)),
        mesh=mesh,
        compiler_params=pltpu.CompilerParams(use_tc_tiling_on_sc=False),
        scratch_types=[
            pltpu.VMEM((chunk,), jnp.int32),
            pltpu.VMEM((chunk, D), jnp.float32),
            pltpu.VMEM((chunk, D), jnp.float32),
            pltpu.VMEM((rows, 128), jnp.float32),
            pltpu.VMEM((rows, 128), jnp.float32),
            pltpu.VMEM((rows, 128), jnp.float32),
            pltpu.VMEM((rows, 128), jnp.float32),
            pltpu.SemaphoreType.DMA,
            pltpu.SemaphoreType.DMA,
        ],
    )
    def gather_kernel(idx_hbm, emb_hbm, cov_hbm,
                      h_out, c_out, sf_out, s0_out,
                      idx_v, e_scr, c_scr, h_p, c_p, sf_p, s0_p,
                      sem_e, sem_c):
        wid = lax.axis_index("s") * NC + lax.axis_index("c")
        base = wid * per_w

        @pl.loop(0, n_chunks)
        def _chunk(k):
            start = base + k * chunk
            pltpu.sync_copy(idx_hbm.at[pl.ds(start, chunk)], idx_v)
            ce = pltpu.async_copy(emb_hbm.at[idx_v], e_scr, sem_e)
            cc = pltpu.async_copy(cov_hbm.at[idx_v], c_scr, sem_c)
            ce.wait()
            cc.wait()
            for j in range(chunk):
                r, g = j // 4, 32 * (j % 4)
                h_p[r, pl.ds(g, 16)] = e_scr[j, pl.ds(0, 16)]
                h_p[r, pl.ds(g + 16, 16)] = e_scr[j, pl.ds(16, 16)]
                c_p[r, pl.ds(g, 16)] = c_scr[j, pl.ds(0, 16)]
                c_p[r, pl.ds(g + 16, 16)] = c_scr[j, pl.ds(16, 16)]
            for b in range(nb):
                j0 = b * S
                sa = e_scr[j0, pl.ds(0, 16)]
                sb = e_scr[j0, pl.ds(16, 16)]
                ca = c_scr[j0, pl.ds(0, 16)]
                cb = c_scr[j0, pl.ds(16, 16)]
                for q in range(5):
                    rr = 5 * b + q
                    for g in range(4):
                        sf_p[rr, pl.ds(32 * g, 16)] = sa
                        sf_p[rr, pl.ds(32 * g + 16, 16)] = sb
                        s0_p[rr, pl.ds(32 * g, 16)] = ca
                        s0_p[rr, pl.ds(32 * g + 16, 16)] = cb
            orow = base // 4 + k * rows
            pltpu.sync_copy(h_p, h_out.at[pl.ds(orow, rows)])
            pltpu.sync_copy(c_p, c_out.at[pl.ds(orow, rows)])
            pltpu.sync_copy(sf_p, sf_out.at[pl.ds(orow, rows)])
            pltpu.sync_copy(s0_p, s0_out.at[pl.ds(orow, rows)])

    return gather_kernel(idx_flat, emb40, cov)


def _math_body(h_ref, c_ref, sf_ref, s0_ref, o_ref):
    f32 = jnp.float32
    h = h_ref[...]
    cv = c_ref[...]
    sf = sf_ref[...]
    s0r = s0_ref[...]

    seg = (lax.broadcasted_iota(jnp.int32, (128, 4), 0) // 32
           == lax.broadcasted_iota(jnp.int32, (128, 4), 1)).astype(f32)

    def segsum(x):
        return lax.dot_general(x, seg, (((1,), (0,)), ((), ())),
                               preferred_element_type=f32,
                               precision=lax.Precision.HIGHEST)

    a = 1.0 - EPS
    sig = jnp.where(cv > 0, cv, a * (jnp.exp(cv) - 1.0)) + 1.0
    sig = jnp.maximum(sig, EPS)
    sig0 = jnp.where(s0r > 0, s0r, a * (jnp.exp(s0r) - 1.0)) + 1.0
    sig0 = jnp.maximum(sig0, EPS)
    r0 = 1.0 / sig0

    dots_h = segsum(h * sf)
    hh = segsum(h * h)
    ss = segsum(sf * sf)
    trace = segsum(sig * r0)
    uu_a = segsum(h * h * r0)
    uu_b = segsum(h * sf * r0)
    uu_c = segsum(sf * sf * r0)
    ld_t = segsum(jnp.log(sig))
    ld_s = segsum(jnp.log(sig0))

    t_t = jnp.sqrt(1.0 + hh)
    t_s = jnp.sqrt(1.0 + ss)

    alpha = t_t * t_s - dots_h
    alpha = 1.0 + jnp.maximum(alpha - 1.0, EPS)
    sq = jnp.sqrt(jnp.maximum(alpha * alpha - 1.0, 0.0))
    denom = jnp.maximum(sq, EPS)
    cfac = jnp.log(alpha + sq) / denom
    beta = t_s
    mdot = (-beta * cfac * (dots_h - alpha * ss)
            - (1.0 - beta * t_s) * cfac * (t_t - alpha * t_s))
    scale = mdot / jnp.maximum(beta + 1.0, EPS)
    mu = scale - cfac * alpha
    uu = cfac * cfac * uu_a + 2.0 * cfac * mu * uu_b + mu * mu * uu_c
    kds = 0.5 * (trace + uu - D - (ld_t - ld_s))
    o_ref[...] = kds


def _tc_math(h, c, sf, s0, bB=256, interpret=False):
    R = h.shape[0]
    rB = bB * S // 4
    return pl.pallas_call(
        _math_body,
        grid=(R // rB,),
        in_specs=[pl.BlockSpec((rB, 128), lambda i: (i, 0)),
                  pl.BlockSpec((rB, 128), lambda i: (i, 0)),
                  pl.BlockSpec((rB, 128), lambda i: (i, 0)),
                  pl.BlockSpec((rB, 128), lambda i: (i, 0))],
        out_specs=pl.BlockSpec((rB, 4), lambda i: (i, 0)),
        out_shape=jax.ShapeDtypeStruct((R, 4), jnp.float32),
        interpret=interpret,
    )(h, c, sf, s0)


def kernel(idx, embedding, covariance):
    B, S_ = idx.shape
    n = B * S_
    idx_flat = idx.reshape(-1)
    emb_head = embedding[:, :D]
    h, c, sf, s0 = _sc_gather_pack(idx_flat, emb_head, covariance, n)
    kds4 = _tc_math(h, c, sf, s0)
    return kds4.reshape(B, S_)[:, 1:]
